# Initial kernel scaffold; baseline (speedup 1.0000x reference)
import jax
import jax.numpy as jnp
from jax.experimental import pallas as pl


def kernel(X):
    raise NotImplementedError("write your pallas kernel here")


# speedup vs baseline: 64.5688x; 64.5688x over previous
import jax
import jax.numpy as jnp
from jax.experimental import pallas as pl

_TOP_K = 256
_STEP = 32
_VALUE = -10000000.0
_ROWS_PER_TILE = 128


def _tile_body(xa_ref, xw_ref, o_ref):
    qt = pl.program_id(1)

    @pl.when(qt == 0)
    def _sliding():
        x = xa_ref[0]
        r = jax.lax.broadcasted_iota(jnp.int32, x.shape, 0)
        c = jax.lax.broadcasted_iota(jnp.int32, x.shape, 1)
        start = jnp.minimum(r * _STEP, _STEP * 119)
        mask = (c >= start) & (c < start + _TOP_K)
        xm = jnp.where(mask, x, jnp.float32(_VALUE))
        m = jnp.max(xm, axis=-1, keepdims=True)
        e = jnp.exp(xm - m)
        s = jnp.sum(e, axis=-1, keepdims=True)
        o_ref[0] = e / s

    @pl.when(qt != 0)
    def _fixed():
        w = xw_ref[0]
        m = jnp.max(w, axis=-1, keepdims=True)
        e = jnp.exp(w - m)
        s = jnp.sum(e, axis=-1, keepdims=True)
        y = e / s
        rows = o_ref.shape[1]
        left = jnp.zeros((rows, _STEP * 119), jnp.float32)
        right = jnp.zeros((rows, o_ref.shape[2] - _STEP * 119 - _TOP_K), jnp.float32)
        o_ref[0] = jnp.concatenate([left, y, right], axis=-1)


def kernel(X):
    B, Q, K = X.shape
    fixed_start = _STEP * 119
    Xw = X[:, :, fixed_start:fixed_start + _TOP_K]
    Xa = X[:, :_ROWS_PER_TILE, :]
    n_qt = Q // _ROWS_PER_TILE
    return pl.pallas_call(
        _tile_body,
        grid=(B, n_qt),
        in_specs=[
            pl.BlockSpec((1, _ROWS_PER_TILE, K), lambda b, q: (b, 0, 0)),
            pl.BlockSpec((1, _ROWS_PER_TILE, _TOP_K), lambda b, q: (b, q, 0)),
        ],
        out_specs=pl.BlockSpec((1, _ROWS_PER_TILE, K), lambda b, q: (b, q, 0)),
        out_shape=jax.ShapeDtypeStruct((B, Q, K), jnp.float32),
    )(Xa, Xw)
